# baseline (speedup 1.0000x reference)
import functools
import math

import jax
import jax.numpy as jnp
from jax import lax
from jax.experimental import pallas as pl
from jax.experimental.pallas import tpu as pltpu
from jax.experimental.pallas import tpu_sc as plsc

B, T, D, H, E, K = 1, 2048, 1024, 16, 8, 2
DH = D // H
FF = 4 * D
N = B * T
CAP = N * K // E
EP = 128
NEG = -1e30

BT = 256
BQ = 256
BK = 256
BFF = 1024


def _k1_body(x_ref, g_ref, b_ref, wqkv_ref, cos_ref, sin_ref,
             q_ref, k_ref, v_ref):
    x = x_ref[...]
    mu = jnp.mean(x, axis=1, keepdims=True)
    var = jnp.mean((x - mu) ** 2, axis=1, keepdims=True)
    h = (x - mu) / jnp.sqrt(var + 1e-5) * g_ref[...] + b_ref[...]
    qkv = jnp.dot(h, wqkv_ref[...], preferred_element_type=jnp.float32)
    q = qkv[:, :D]
    k = qkv[:, D:2 * D]
    v = qkv[:, 2 * D:]
    cos = cos_ref[...]
    sin = sin_ref[...]
    lane = jax.lax.broadcasted_iota(jnp.int32, (BT, D), 1)
    first_half = (lane % DH) < (DH // 2)

    def rot(a):
        a_sw = jnp.where(first_half,
                         jnp.roll(a, -DH // 2, axis=1),
                         jnp.roll(a, DH // 2, axis=1))
        return a * cos + a_sw * sin

    q_ref[...] = rot(q)
    k_ref[...] = rot(k)
    v_ref[...] = v


def _k2_body(q_ref, k_ref, v_ref, o_ref):
    qb = pl.program_id(1)
    scale = 1.0 / math.sqrt(DH)
    rows = qb * BQ + jax.lax.broadcasted_iota(jnp.int32, (BQ, BK), 0)

    for sub in range(2):
        q = q_ref[:, sub * DH:(sub + 1) * DH] * scale

        def body(kb, carry):
            m, l, acc = carry
            kk = k_ref[pl.ds(kb * BK, BK), sub * DH:(sub + 1) * DH]
            vv = v_ref[pl.ds(kb * BK, BK), sub * DH:(sub + 1) * DH]
            s = jax.lax.dot_general(q, kk, (((1,), (1,)), ((), ())),
                                    preferred_element_type=jnp.float32)
            cols = kb * BK + jax.lax.broadcasted_iota(jnp.int32, (BQ, BK), 1)
            s = jnp.where(rows >= cols, s, NEG)
            m_new = jnp.maximum(m, jnp.max(s, axis=1, keepdims=True))
            p = jnp.exp(s - m_new)
            corr = jnp.exp(m - m_new)
            l_new = l * corr + jnp.sum(p, axis=1, keepdims=True)
            acc_new = acc * corr + jnp.dot(p, vv,
                                           preferred_element_type=jnp.float32)
            return m_new, l_new, acc_new

        m0 = jnp.full((BQ, 1), NEG, jnp.float32)
        l0 = jnp.zeros((BQ, 1), jnp.float32)
        a0 = jnp.zeros((BQ, DH), jnp.float32)
        m, l, acc = jax.lax.fori_loop(0, qb + 1, body, (m0, l0, a0))
        o_ref[sub] = acc / l


def _k3_body(x_ref, ctx_ref, wproj_ref, g_ref, b_ref,
             wr_ref, br_ref, wn_ref, bn_ref, eps_ref,
             x1_ref, h2_ref, noisy_ref):
    x1 = x_ref[...] + jnp.dot(ctx_ref[...], wproj_ref[...],
                              preferred_element_type=jnp.float32)
    x1_ref[...] = x1
    mu = jnp.mean(x1, axis=1, keepdims=True)
    var = jnp.mean((x1 - mu) ** 2, axis=1, keepdims=True)
    h2 = (x1 - mu) / jnp.sqrt(var + 1e-5) * g_ref[...] + b_ref[...]
    h2_ref[...] = h2
    logits = jnp.dot(h2, wr_ref[...], preferred_element_type=jnp.float32) + br_ref[...]
    pre = jnp.dot(h2, wn_ref[...], preferred_element_type=jnp.float32) + bn_ref[...]
    noise = jnp.maximum(pre, 0.0) + jnp.log1p(jnp.exp(-jnp.abs(pre)))
    noisy_ref[...] = logits + eps_ref[...] * noise


def _top2(nz, rows_n):
    lane = jax.lax.broadcasted_iota(jnp.int32, (rows_n, EP), 1)
    v0 = jnp.max(nz, axis=1, keepdims=True)
    e0 = jnp.min(jnp.where(nz == v0, lane, EP), axis=1, keepdims=True)
    nz1 = jnp.where(lane == e0, NEG, nz)
    v1 = jnp.max(nz1, axis=1, keepdims=True)
    e1 = jnp.min(jnp.where(nz1 == v1, lane, EP), axis=1, keepdims=True)
    is0 = (lane == e0)
    is1 = (lane == e1)
    mask = jnp.where(is0 | is1, 1.0, 0.0)
    ev = jnp.exp(v1 - v0)
    g0 = 1.0 / (1.0 + ev)
    g1 = ev / (1.0 + ev)
    return mask, is0, is1, g0, g1, lane


def _k4_body(noisy_ref, pertok_ref, idxf_ref):
    pid = pl.program_id(0)
    BR = N // 16
    rstart = pid * BR

    mask, _, _, _, _, _ = _top2(noisy_ref[...], N)
    _, is0, is1, g0, g1, lane = _top2(noisy_ref[pl.ds(rstart, BR), :], BR)

    rows = rstart + jax.lax.broadcasted_iota(jnp.int32, (BR, N), 0)
    tcols = jax.lax.broadcasted_iota(jnp.int32, (BR, N), 1)
    lt = jnp.where(tcols < rows, 1.0, 0.0)
    rank = jnp.dot(lt, mask, preferred_element_type=jnp.float32)
    lane_f = lane.astype(jnp.float32)

    e0v = jnp.sum(jnp.where(is0, lane_f, 0.0), axis=1, keepdims=True)
    e1v = jnp.sum(jnp.where(is1, lane_f, 0.0), axis=1, keepdims=True)
    s0 = jnp.sum(jnp.where(is0, rank, 0.0), axis=1, keepdims=True)
    s1 = jnp.sum(jnp.where(is1, rank, 0.0), axis=1, keepdims=True)
    ok0 = s0 < CAP
    ok1 = s1 < CAP
    f0 = jnp.where(ok0, e0v * CAP + s0, 0.0)
    f1 = jnp.where(ok1, e1v * CAP + s1, 0.0)
    g0v = jnp.where(ok0, g0, 0.0)
    g1v = jnp.where(ok1, g1, 0.0)
    pertok_ref[...] = (jnp.where(lane == 0, f0, 0.0)
                       + jnp.where(lane == 1, f1, 0.0)
                       + jnp.where(lane == 2, g0v, 0.0)
                       + jnp.where(lane == 3, g1v, 0.0))

    @pl.when(pid == 0)
    def _():
        idxf_ref[...] = jnp.zeros((E, 1, CAP), jnp.float32)

    rank_sel = jnp.where((is0 | is1), rank, -1.0)
    r_iota = jax.lax.broadcasted_iota(jnp.int32, (BR, CAP), 1)
    tok_row = (rstart + jax.lax.broadcasted_iota(
        jnp.int32, (1, BR), 1)).astype(jnp.float32)
    for e in range(E):
        col = rank_sel[:, e:e + 1]
        a = jnp.where(col.astype(jnp.int32) == r_iota, 1.0, 0.0)
        idxf_ref[e, 0] += jnp.dot(tok_row, a,
                                  preferred_element_type=jnp.float32,
                                  precision=jax.lax.Precision.HIGHEST)[0]


def _k5_body(xin_ref, w1_ref, b1_ref, w2_ref, b2_ref, oexp_ref, acc_scr):
    ffb = pl.program_id(1)

    @pl.when(ffb == 0)
    def _():
        acc_scr[...] = jnp.zeros((CAP, D), jnp.float32)

    mid = jnp.maximum(
        jnp.dot(xin_ref[...], w1_ref[0], preferred_element_type=jnp.float32)
        + b1_ref[0], 0.0)
    acc_scr[...] += jnp.dot(mid, w2_ref[0], preferred_element_type=jnp.float32)

    @pl.when(ffb == FF // BFF - 1)
    def _():
        oexp_ref[0] = acc_scr[...] + b2_ref[0]


def _make_sc_gather(rows_total, table_rows):
    info = plsc.get_sparse_core_info()
    nw = info.num_cores * info.num_subcores
    per_w = rows_total // nw
    chunk = min(64, per_w)
    n_iter = per_w // chunk
    mesh = plsc.VectorSubcoreMesh(core_axis_name="c", subcore_axis_name="s")

    @functools.partial(
        pl.kernel, mesh=mesh,
        out_type=jax.ShapeDtypeStruct((rows_total, D), jnp.float32),
        scratch_types=[pltpu.VMEM((chunk,), jnp.int32),
                       pltpu.VMEM((chunk, D), jnp.float32),
                       pltpu.SemaphoreType.DMA],
    )
    def g(table_hbm, idx_hbm, out_hbm, idx_v, rows_v, sem):
        wid = lax.axis_index("s") * info.num_cores + lax.axis_index("c")
        for c in range(n_iter):
            base = wid * per_w + c * chunk
            pltpu.sync_copy(idx_hbm.at[pl.ds(base, chunk)], idx_v)
            pltpu.async_copy(table_hbm.at[idx_v], rows_v, sem).wait()
            pltpu.sync_copy(rows_v, out_hbm.at[pl.ds(base, chunk)])

    return g


def _k6_body(pertok_ref, u0_ref, u1_ref, x1_ref, out_ref):
    g0 = pertok_ref[:, 2:3]
    g1 = pertok_ref[:, 3:4]
    out_ref[...] = x1_ref[...] + g0 * u0_ref[...] + g1 * u1_ref[...]


def kernel(x, Wqkv, Wproj, ln1_g, ln1_b, ln2_g, ln2_b, Wr, br, Wn, bn,
           We1, be1, We2, be2):
    f32 = jnp.float32
    x2 = x.reshape(N, D)

    half = DH // 2
    pos = jnp.arange(T, dtype=f32)[:, None]
    inv = jnp.exp(jnp.arange(0, DH, 2, dtype=f32) * (-math.log(10000.0) / DH))
    ang = pos * inv
    cos1 = jnp.cos(ang)
    sin1 = jnp.sin(ang)
    cos_full = jnp.tile(jnp.concatenate([cos1, cos1], axis=1), (1, H))
    sin_full = jnp.tile(jnp.concatenate([-sin1, sin1], axis=1), (1, H))

    eps = jax.random.normal(jax.random.key(42), (B, T, E), dtype=f32)
    eps_p = jnp.zeros((N, EP), f32).at[:, :E].set(eps.reshape(N, E))
    Wr_p = jnp.zeros((D, EP), f32).at[:, :E].set(Wr)
    Wn_p = jnp.zeros((D, EP), f32).at[:, :E].set(Wn)
    br_p = jnp.full((1, EP), NEG, f32).at[0, :E].set(br)
    bn_p = jnp.zeros((1, EP), f32).at[0, :E].set(bn)
    ln1g = ln1_g.reshape(1, D)
    ln1b = ln1_b.reshape(1, D)
    ln2g = ln2_g.reshape(1, D)
    ln2b = ln2_b.reshape(1, D)
    be1_3 = be1.reshape(E, 1, FF)
    be2_3 = be2.reshape(E, 1, D)

    row_spec = pl.BlockSpec((BT, D), lambda i: (i, 0))
    vec_spec = pl.BlockSpec((1, D), lambda i: (0, 0))
    q, k, v = pl.pallas_call(
        _k1_body,
        grid=(N // BT,),
        in_specs=[row_spec, vec_spec, vec_spec,
                  pl.BlockSpec((D, 3 * D), lambda i: (0, 0)),
                  row_spec, row_spec],
        out_specs=[row_spec, row_spec, row_spec],
        out_shape=[jax.ShapeDtypeStruct((N, D), f32)] * 3,
    )(x2, ln1g, ln1b, Wqkv, cos_full, sin_full)

    ctx = pl.pallas_call(
        _k2_body,
        grid=(H // 2, N // BQ),
        in_specs=[pl.BlockSpec((BQ, 2 * DH), lambda hp, qb: (qb, hp)),
                  pl.BlockSpec((N, 2 * DH), lambda hp, qb: (0, hp)),
                  pl.BlockSpec((N, 2 * DH), lambda hp, qb: (0, hp))],
        out_specs=pl.BlockSpec((2, BQ, DH), lambda hp, qb: (hp, qb, 0)),
        out_shape=jax.ShapeDtypeStruct((H, T, DH), f32),
    )(q, k, v)
    ctx = ctx.reshape(N, D)

    ep_spec = pl.BlockSpec((BT, EP), lambda i: (i, 0))
    ep_vec = pl.BlockSpec((1, EP), lambda i: (0, 0))
    x1, h2, noisy = pl.pallas_call(
        _k3_body,
        grid=(N // BT,),
        in_specs=[row_spec, row_spec,
                  pl.BlockSpec((D, D), lambda i: (0, 0)),
                  vec_spec, vec_spec,
                  pl.BlockSpec((D, EP), lambda i: (0, 0)), ep_vec,
                  pl.BlockSpec((D, EP), lambda i: (0, 0)), ep_vec,
                  ep_spec],
        out_specs=[row_spec, row_spec, ep_spec],
        out_shape=[jax.ShapeDtypeStruct((N, D), f32),
                   jax.ShapeDtypeStruct((N, D), f32),
                   jax.ShapeDtypeStruct((N, EP), f32)],
    )(x2, ctx, Wproj, ln2g, ln2b, Wr_p, br_p, Wn_p, bn_p, eps_p)

    BR = N // 16
    pertok, idxf = pl.pallas_call(
        _k4_body,
        grid=(16,),
        in_specs=[pl.BlockSpec((N, EP), lambda i: (0, 0))],
        out_specs=[pl.BlockSpec((BR, EP), lambda i: (i, 0)),
                   pl.BlockSpec((E, 1, CAP), lambda i: (0, 0, 0))],
        out_shape=[jax.ShapeDtypeStruct((N, EP), f32),
                   jax.ShapeDtypeStruct((E, 1, CAP), f32)],
    )(noisy)

    disp_idx = idxf.reshape(E * CAP).astype(jnp.int32)
    flat01 = jnp.concatenate(
        [pertok[:, 0], pertok[:, 1]]).astype(jnp.int32)

    xin = _make_sc_gather(E * CAP, N)(h2, disp_idx)

    oexp = pl.pallas_call(
        _k5_body,
        grid=(E, FF // BFF),
        in_specs=[pl.BlockSpec((CAP, D), lambda e, f: (e, 0)),
                  pl.BlockSpec((1, D, BFF), lambda e, f: (e, 0, f)),
                  pl.BlockSpec((1, 1, BFF), lambda e, f: (e, 0, f)),
                  pl.BlockSpec((1, BFF, D), lambda e, f: (e, f, 0)),
                  pl.BlockSpec((1, 1, D), lambda e, f: (e, 0, 0))],
        out_specs=pl.BlockSpec((1, CAP, D), lambda e, f: (e, 0, 0)),
        out_shape=jax.ShapeDtypeStruct((E, CAP, D), f32),
        scratch_shapes=[pltpu.VMEM((CAP, D), f32)],
    )(xin, We1, be1_3, We2, be2_3)

    u01 = _make_sc_gather(2 * N, E * CAP)(oexp.reshape(E * CAP, D), flat01)

    out = pl.pallas_call(
        _k6_body,
        grid=(N // BT,),
        in_specs=[pl.BlockSpec((BT, EP), lambda i: (i, 0)),
                  pl.BlockSpec((BT, D), lambda i: (i, 0)),
                  pl.BlockSpec((BT, D), lambda i: (i + N // BT, 0)),
                  pl.BlockSpec((BT, D), lambda i: (i, 0))],
        out_specs=pl.BlockSpec((BT, D), lambda i: (i, 0)),
        out_shape=jax.ShapeDtypeStruct((N, D), f32),
    )(pertok, u01, u01, x1)

    return out.reshape(B, T, D)


# speedup vs baseline: 1.3658x; 1.3658x over previous
import functools
import math

import jax
import jax.numpy as jnp
from jax import lax
from jax.experimental import pallas as pl
from jax.experimental.pallas import tpu as pltpu
from jax.experimental.pallas import tpu_sc as plsc

B, T, D, H, E, K = 1, 2048, 1024, 16, 8, 2
DH = D // H
FF = 4 * D
N = B * T
CAP = N * K // E
EP = 128
NEG = -1e30

BT = 256
BQ = 512
BK = 512
BFF = 1024


def _k1_body(x_ref, g_ref, b_ref, wqkv_ref, cos_ref, sin_ref,
             q_ref, k_ref, v_ref):
    x = x_ref[...]
    mu = jnp.mean(x, axis=1, keepdims=True)
    var = jnp.mean((x - mu) ** 2, axis=1, keepdims=True)
    h = (x - mu) / jnp.sqrt(var + 1e-5) * g_ref[...] + b_ref[...]
    qkv = jnp.dot(h, wqkv_ref[...], preferred_element_type=jnp.float32)
    q = qkv[:, :D]
    k = qkv[:, D:2 * D]
    v = qkv[:, 2 * D:]
    cos = cos_ref[...]
    sin = sin_ref[...]
    lane = jax.lax.broadcasted_iota(jnp.int32, (BT, D), 1)
    first_half = (lane % DH) < (DH // 2)

    def rot(a):
        a_sw = jnp.where(first_half,
                         jnp.roll(a, -DH // 2, axis=1),
                         jnp.roll(a, DH // 2, axis=1))
        return a * cos + a_sw * sin

    q_ref[...] = rot(q)
    k_ref[...] = rot(k)
    v_ref[...] = v


def _k2_body(q_ref, k_ref, v_ref, o_ref):
    qb = pl.program_id(1)
    scale = 1.0 / math.sqrt(DH)
    rows = qb * BQ + jax.lax.broadcasted_iota(jnp.int32, (BQ, BK), 0)

    for sub in range(2):
        q = q_ref[:, sub * DH:(sub + 1) * DH] * scale

        def body(kb, carry):
            m, l, acc = carry
            kk = k_ref[pl.ds(kb * BK, BK), sub * DH:(sub + 1) * DH]
            vv = v_ref[pl.ds(kb * BK, BK), sub * DH:(sub + 1) * DH]
            s = jax.lax.dot_general(q, kk, (((1,), (1,)), ((), ())),
                                    preferred_element_type=jnp.float32)
            cols = kb * BK + jax.lax.broadcasted_iota(jnp.int32, (BQ, BK), 1)
            s = jnp.where(rows >= cols, s, NEG)
            m_new = jnp.maximum(m, jnp.max(s, axis=1, keepdims=True))
            p = jnp.exp(s - m_new)
            corr = jnp.exp(m - m_new)
            l_new = l * corr + jnp.sum(p, axis=1, keepdims=True)
            acc_new = acc * corr + jnp.dot(p, vv,
                                           preferred_element_type=jnp.float32)
            return m_new, l_new, acc_new

        m0 = jnp.full((BQ, 1), NEG, jnp.float32)
        l0 = jnp.zeros((BQ, 1), jnp.float32)
        a0 = jnp.zeros((BQ, DH), jnp.float32)
        m, l, acc = jax.lax.fori_loop(0, qb + 1, body, (m0, l0, a0))
        o_ref[sub] = acc / l


def _k3_body(x_ref, ctx_ref, wproj_ref, g_ref, b_ref,
             wr_ref, br_ref, wn_ref, bn_ref, eps_ref,
             x1_ref, h2_ref, noisy_ref):
    x1 = x_ref[...] + jnp.dot(ctx_ref[...], wproj_ref[...],
                              preferred_element_type=jnp.float32)
    x1_ref[...] = x1
    mu = jnp.mean(x1, axis=1, keepdims=True)
    var = jnp.mean((x1 - mu) ** 2, axis=1, keepdims=True)
    h2 = (x1 - mu) / jnp.sqrt(var + 1e-5) * g_ref[...] + b_ref[...]
    h2_ref[...] = h2
    logits = jnp.dot(h2, wr_ref[...], preferred_element_type=jnp.float32) + br_ref[...]
    pre = jnp.dot(h2, wn_ref[...], preferred_element_type=jnp.float32) + bn_ref[...]
    noise = jnp.maximum(pre, 0.0) + jnp.log1p(jnp.exp(-jnp.abs(pre)))
    noisy_ref[...] = logits + eps_ref[...] * noise


def _top2(nz, rows_n):
    lane = jax.lax.broadcasted_iota(jnp.int32, (rows_n, EP), 1)
    v0 = jnp.max(nz, axis=1, keepdims=True)
    e0 = jnp.min(jnp.where(nz == v0, lane, EP), axis=1, keepdims=True)
    nz1 = jnp.where(lane == e0, NEG, nz)
    v1 = jnp.max(nz1, axis=1, keepdims=True)
    e1 = jnp.min(jnp.where(nz1 == v1, lane, EP), axis=1, keepdims=True)
    is0 = (lane == e0)
    is1 = (lane == e1)
    mask = jnp.where(is0 | is1, 1.0, 0.0)
    ev = jnp.exp(v1 - v0)
    g0 = 1.0 / (1.0 + ev)
    g1 = ev / (1.0 + ev)
    return mask, is0, is1, g0, g1, lane


def _k4_body(noisy_ref, pertok_ref, idxf_ref):
    pid = pl.program_id(0)
    BR = N // 16
    rstart = pid * BR

    mask, _, _, _, _, _ = _top2(noisy_ref[...], N)
    _, is0, is1, g0, g1, lane = _top2(noisy_ref[pl.ds(rstart, BR), :], BR)

    rows = rstart + jax.lax.broadcasted_iota(jnp.int32, (BR, N), 0)
    tcols = jax.lax.broadcasted_iota(jnp.int32, (BR, N), 1)
    lt = jnp.where(tcols < rows, 1.0, 0.0)
    rank = jnp.dot(lt, mask, preferred_element_type=jnp.float32)
    lane_f = lane.astype(jnp.float32)

    e0v = jnp.sum(jnp.where(is0, lane_f, 0.0), axis=1, keepdims=True)
    e1v = jnp.sum(jnp.where(is1, lane_f, 0.0), axis=1, keepdims=True)
    s0 = jnp.sum(jnp.where(is0, rank, 0.0), axis=1, keepdims=True)
    s1 = jnp.sum(jnp.where(is1, rank, 0.0), axis=1, keepdims=True)
    ok0 = s0 < CAP
    ok1 = s1 < CAP
    f0 = jnp.where(ok0, e0v * CAP + s0, 0.0)
    f1 = jnp.where(ok1, e1v * CAP + s1, 0.0)
    g0v = jnp.where(ok0, g0, 0.0)
    g1v = jnp.where(ok1, g1, 0.0)
    pertok_ref[...] = (jnp.where(lane == 0, f0, 0.0)
                       + jnp.where(lane == 1, f1, 0.0)
                       + jnp.where(lane == 2, g0v, 0.0)
                       + jnp.where(lane == 3, g1v, 0.0))

    @pl.when(pid == 0)
    def _():
        idxf_ref[...] = jnp.zeros((E, 1, CAP), jnp.float32)

    rank_sel = jnp.where((is0 | is1), rank, -1.0)
    r_iota = jax.lax.broadcasted_iota(jnp.int32, (BR, CAP), 1)
    tok_row = (rstart + jax.lax.broadcasted_iota(
        jnp.int32, (1, BR), 1)).astype(jnp.float32)
    for e in range(E):
        col = rank_sel[:, e:e + 1]
        a = jnp.where(col.astype(jnp.int32) == r_iota, 1.0, 0.0)
        idxf_ref[e, 0] += jnp.dot(tok_row, a,
                                  preferred_element_type=jnp.float32,
                                  precision=jax.lax.Precision.HIGHEST)[0]


def _k5_body(xin_ref, w1_ref, b1_ref, w2_ref, b2_ref, oexp_ref, acc_scr):
    ffb = pl.program_id(1)

    @pl.when(ffb == 0)
    def _():
        acc_scr[...] = jnp.zeros((CAP, D), jnp.float32)

    mid = jnp.maximum(
        jnp.dot(xin_ref[...], w1_ref[0], preferred_element_type=jnp.float32)
        + b1_ref[0], 0.0)
    acc_scr[...] += jnp.dot(mid, w2_ref[0], preferred_element_type=jnp.float32)

    @pl.when(ffb == FF // BFF - 1)
    def _():
        oexp_ref[0] = acc_scr[...] + b2_ref[0]


def _make_sc_gather(rows_total, table_rows):
    info = plsc.get_sparse_core_info()
    nw = info.num_cores * info.num_subcores
    per_w = rows_total // nw
    chunk = min(64, per_w)
    n_iter = per_w // chunk
    mesh = plsc.VectorSubcoreMesh(core_axis_name="c", subcore_axis_name="s")

    @functools.partial(
        pl.kernel, mesh=mesh,
        out_type=jax.ShapeDtypeStruct((rows_total, D), jnp.float32),
        scratch_types=[pltpu.VMEM((chunk,), jnp.int32),
                       pltpu.VMEM((chunk, D), jnp.float32),
                       pltpu.SemaphoreType.DMA],
    )
    def g(table_hbm, idx_hbm, out_hbm, idx_v, rows_v, sem):
        wid = lax.axis_index("s") * info.num_cores + lax.axis_index("c")
        for c in range(n_iter):
            base = wid * per_w + c * chunk
            pltpu.sync_copy(idx_hbm.at[pl.ds(base, chunk)], idx_v)
            pltpu.async_copy(table_hbm.at[idx_v], rows_v, sem).wait()
            pltpu.sync_copy(rows_v, out_hbm.at[pl.ds(base, chunk)])

    return g


def _k6_body(pertok_ref, u0_ref, u1_ref, x1_ref, out_ref):
    g0 = pertok_ref[:, 2:3]
    g1 = pertok_ref[:, 3:4]
    out_ref[...] = x1_ref[...] + g0 * u0_ref[...] + g1 * u1_ref[...]


def kernel(x, Wqkv, Wproj, ln1_g, ln1_b, ln2_g, ln2_b, Wr, br, Wn, bn,
           We1, be1, We2, be2):
    f32 = jnp.float32
    x2 = x.reshape(N, D)

    half = DH // 2
    pos = jnp.arange(T, dtype=f32)[:, None]
    inv = jnp.exp(jnp.arange(0, DH, 2, dtype=f32) * (-math.log(10000.0) / DH))
    ang = pos * inv
    cos1 = jnp.cos(ang)
    sin1 = jnp.sin(ang)
    cos_full = jnp.tile(jnp.concatenate([cos1, cos1], axis=1), (1, H))
    sin_full = jnp.tile(jnp.concatenate([-sin1, sin1], axis=1), (1, H))

    eps = jax.random.normal(jax.random.key(42), (B, T, E), dtype=f32)
    eps_p = jnp.zeros((N, EP), f32).at[:, :E].set(eps.reshape(N, E))
    Wr_p = jnp.zeros((D, EP), f32).at[:, :E].set(Wr)
    Wn_p = jnp.zeros((D, EP), f32).at[:, :E].set(Wn)
    br_p = jnp.full((1, EP), NEG, f32).at[0, :E].set(br)
    bn_p = jnp.zeros((1, EP), f32).at[0, :E].set(bn)
    ln1g = ln1_g.reshape(1, D)
    ln1b = ln1_b.reshape(1, D)
    ln2g = ln2_g.reshape(1, D)
    ln2b = ln2_b.reshape(1, D)
    be1_3 = be1.reshape(E, 1, FF)
    be2_3 = be2.reshape(E, 1, D)

    row_spec = pl.BlockSpec((BT, D), lambda i: (i, 0))
    vec_spec = pl.BlockSpec((1, D), lambda i: (0, 0))
    q, k, v = pl.pallas_call(
        _k1_body,
        grid=(N // BT,),
        in_specs=[row_spec, vec_spec, vec_spec,
                  pl.BlockSpec((D, 3 * D), lambda i: (0, 0)),
                  row_spec, row_spec],
        out_specs=[row_spec, row_spec, row_spec],
        out_shape=[jax.ShapeDtypeStruct((N, D), f32)] * 3,
    )(x2, ln1g, ln1b, Wqkv, cos_full, sin_full)

    ctx = pl.pallas_call(
        _k2_body,
        grid=(H // 2, N // BQ),
        in_specs=[pl.BlockSpec((BQ, 2 * DH), lambda hp, qb: (qb, hp)),
                  pl.BlockSpec((N, 2 * DH), lambda hp, qb: (0, hp)),
                  pl.BlockSpec((N, 2 * DH), lambda hp, qb: (0, hp))],
        out_specs=pl.BlockSpec((2, BQ, DH), lambda hp, qb: (hp, qb, 0)),
        out_shape=jax.ShapeDtypeStruct((H, T, DH), f32),
    )(q, k, v)
    ctx = ctx.reshape(N, D)

    ep_spec = pl.BlockSpec((BT, EP), lambda i: (i, 0))
    ep_vec = pl.BlockSpec((1, EP), lambda i: (0, 0))
    x1, h2, noisy = pl.pallas_call(
        _k3_body,
        grid=(N // BT,),
        in_specs=[row_spec, row_spec,
                  pl.BlockSpec((D, D), lambda i: (0, 0)),
                  vec_spec, vec_spec,
                  pl.BlockSpec((D, EP), lambda i: (0, 0)), ep_vec,
                  pl.BlockSpec((D, EP), lambda i: (0, 0)), ep_vec,
                  ep_spec],
        out_specs=[row_spec, row_spec, ep_spec],
        out_shape=[jax.ShapeDtypeStruct((N, D), f32),
                   jax.ShapeDtypeStruct((N, D), f32),
                   jax.ShapeDtypeStruct((N, EP), f32)],
    )(x2, ctx, Wproj, ln2g, ln2b, Wr_p, br_p, Wn_p, bn_p, eps_p)

    BR = N // 16
    pertok, idxf = pl.pallas_call(
        _k4_body,
        grid=(16,),
        in_specs=[pl.BlockSpec((N, EP), lambda i: (0, 0))],
        out_specs=[pl.BlockSpec((BR, EP), lambda i: (i, 0)),
                   pl.BlockSpec((E, 1, CAP), lambda i: (0, 0, 0))],
        out_shape=[jax.ShapeDtypeStruct((N, EP), f32),
                   jax.ShapeDtypeStruct((E, 1, CAP), f32)],
    )(noisy)

    disp_idx = idxf.reshape(E * CAP).astype(jnp.int32)
    flat01 = jnp.concatenate(
        [pertok[:, 0], pertok[:, 1]]).astype(jnp.int32)

    xin = _make_sc_gather(E * CAP, N)(h2, disp_idx)

    oexp = pl.pallas_call(
        _k5_body,
        grid=(E, FF // BFF),
        in_specs=[pl.BlockSpec((CAP, D), lambda e, f: (e, 0)),
                  pl.BlockSpec((1, D, BFF), lambda e, f: (e, 0, f)),
                  pl.BlockSpec((1, 1, BFF), lambda e, f: (e, 0, f)),
                  pl.BlockSpec((1, BFF, D), lambda e, f: (e, f, 0)),
                  pl.BlockSpec((1, 1, D), lambda e, f: (e, 0, 0))],
        out_specs=pl.BlockSpec((1, CAP, D), lambda e, f: (e, 0, 0)),
        out_shape=jax.ShapeDtypeStruct((E, CAP, D), f32),
        scratch_shapes=[pltpu.VMEM((CAP, D), f32)],
    )(xin, We1, be1_3, We2, be2_3)

    u01 = _make_sc_gather(2 * N, E * CAP)(oexp.reshape(E * CAP, D), flat01)

    out = pl.pallas_call(
        _k6_body,
        grid=(N // BT,),
        in_specs=[pl.BlockSpec((BT, EP), lambda i: (i, 0)),
                  pl.BlockSpec((BT, D), lambda i: (i, 0)),
                  pl.BlockSpec((BT, D), lambda i: (i + N // BT, 0)),
                  pl.BlockSpec((BT, D), lambda i: (i, 0))],
        out_specs=pl.BlockSpec((BT, D), lambda i: (i, 0)),
        out_shape=jax.ShapeDtypeStruct((N, D), f32),
    )(pertok, u01, u01, x1)

    return out.reshape(B, T, D)


# speedup vs baseline: 1.4323x; 1.0487x over previous
import functools
import math

import jax
import jax.numpy as jnp
from jax import lax
from jax.experimental import pallas as pl
from jax.experimental.pallas import tpu as pltpu
from jax.experimental.pallas import tpu_sc as plsc

B, T, D, H, E, K = 1, 2048, 1024, 16, 8, 2
DH = D // H
FF = 4 * D
N = B * T
CAP = N * K // E
EP = 128
NEG = -1e30

BT = 512
BQ = 1024
BK = 1024
BFF = 1024


def _k1_body(x_ref, g_ref, b_ref, wqkv_ref, cos_ref, sin_ref,
             q_ref, k_ref, v_ref):
    x = x_ref[...]
    mu = jnp.mean(x, axis=1, keepdims=True)
    var = jnp.mean((x - mu) ** 2, axis=1, keepdims=True)
    h = (x - mu) / jnp.sqrt(var + 1e-5) * g_ref[...] + b_ref[...]
    qkv = jnp.dot(h, wqkv_ref[...], preferred_element_type=jnp.float32)
    q = qkv[:, :D]
    k = qkv[:, D:2 * D]
    v = qkv[:, 2 * D:]
    cos = cos_ref[...]
    sin = sin_ref[...]
    lane = jax.lax.broadcasted_iota(jnp.int32, (BT, D), 1)
    first_half = (lane % DH) < (DH // 2)

    def rot(a):
        a_sw = jnp.where(first_half,
                         jnp.roll(a, -DH // 2, axis=1),
                         jnp.roll(a, DH // 2, axis=1))
        return a * cos + a_sw * sin

    q_ref[...] = rot(q)
    k_ref[...] = rot(k)
    v_ref[...] = v


def _k2_body(q_ref, k_ref, v_ref, o_ref):
    qb = pl.program_id(1)
    scale = 1.0 / math.sqrt(DH)
    rows = qb * BQ + jax.lax.broadcasted_iota(jnp.int32, (BQ, BK), 0)

    for sub in range(2):
        q = q_ref[:, sub * DH:(sub + 1) * DH] * scale

        def body(kb, carry):
            m, l, acc = carry
            kk = k_ref[pl.ds(kb * BK, BK), sub * DH:(sub + 1) * DH]
            vv = v_ref[pl.ds(kb * BK, BK), sub * DH:(sub + 1) * DH]
            s = jax.lax.dot_general(q, kk, (((1,), (1,)), ((), ())),
                                    preferred_element_type=jnp.float32)
            cols = kb * BK + jax.lax.broadcasted_iota(jnp.int32, (BQ, BK), 1)
            s = jnp.where(rows >= cols, s, NEG)
            m_new = jnp.maximum(m, jnp.max(s, axis=1, keepdims=True))
            p = jnp.exp(s - m_new)
            corr = jnp.exp(m - m_new)
            l_new = l * corr + jnp.sum(p, axis=1, keepdims=True)
            acc_new = acc * corr + jnp.dot(p, vv,
                                           preferred_element_type=jnp.float32)
            return m_new, l_new, acc_new

        m0 = jnp.full((BQ, 1), NEG, jnp.float32)
        l0 = jnp.zeros((BQ, 1), jnp.float32)
        a0 = jnp.zeros((BQ, DH), jnp.float32)
        m, l, acc = jax.lax.fori_loop(0, qb + 1, body, (m0, l0, a0))
        o_ref[sub] = acc / l


def _k3_body(x_ref, ctx_ref, wproj_ref, g_ref, b_ref,
             wr_ref, br_ref, wn_ref, bn_ref, eps_ref,
             x1_ref, h2_ref, noisy_ref):
    x1 = x_ref[...] + jnp.dot(ctx_ref[...], wproj_ref[...],
                              preferred_element_type=jnp.float32)
    x1_ref[...] = x1
    mu = jnp.mean(x1, axis=1, keepdims=True)
    var = jnp.mean((x1 - mu) ** 2, axis=1, keepdims=True)
    h2 = (x1 - mu) / jnp.sqrt(var + 1e-5) * g_ref[...] + b_ref[...]
    h2_ref[...] = h2
    logits = jnp.dot(h2, wr_ref[...], preferred_element_type=jnp.float32) + br_ref[...]
    pre = jnp.dot(h2, wn_ref[...], preferred_element_type=jnp.float32) + bn_ref[...]
    noise = jnp.maximum(pre, 0.0) + jnp.log1p(jnp.exp(-jnp.abs(pre)))
    noisy_ref[...] = logits + eps_ref[...] * noise


def _top2(nz, rows_n):
    lane = jax.lax.broadcasted_iota(jnp.int32, (rows_n, EP), 1)
    v0 = jnp.max(nz, axis=1, keepdims=True)
    e0 = jnp.min(jnp.where(nz == v0, lane, EP), axis=1, keepdims=True)
    nz1 = jnp.where(lane == e0, NEG, nz)
    v1 = jnp.max(nz1, axis=1, keepdims=True)
    e1 = jnp.min(jnp.where(nz1 == v1, lane, EP), axis=1, keepdims=True)
    is0 = (lane == e0)
    is1 = (lane == e1)
    mask = jnp.where(is0 | is1, 1.0, 0.0)
    ev = jnp.exp(v1 - v0)
    g0 = 1.0 / (1.0 + ev)
    g1 = ev / (1.0 + ev)
    return mask, is0, is1, g0, g1, lane


def _k4_body(noisy_ref, pertok_ref, idxf_ref):
    pid = pl.program_id(0)
    BR = N // 16
    rstart = pid * BR

    mask, _, _, _, _, _ = _top2(noisy_ref[...], N)
    _, is0, is1, g0, g1, lane = _top2(noisy_ref[pl.ds(rstart, BR), :], BR)

    rows = rstart + jax.lax.broadcasted_iota(jnp.int32, (BR, N), 0)
    tcols = jax.lax.broadcasted_iota(jnp.int32, (BR, N), 1)
    lt = jnp.where(tcols < rows, 1.0, 0.0)
    rank = jnp.dot(lt, mask, preferred_element_type=jnp.float32)
    lane_f = lane.astype(jnp.float32)

    e0v = jnp.sum(jnp.where(is0, lane_f, 0.0), axis=1, keepdims=True)
    e1v = jnp.sum(jnp.where(is1, lane_f, 0.0), axis=1, keepdims=True)
    s0 = jnp.sum(jnp.where(is0, rank, 0.0), axis=1, keepdims=True)
    s1 = jnp.sum(jnp.where(is1, rank, 0.0), axis=1, keepdims=True)
    ok0 = s0 < CAP
    ok1 = s1 < CAP
    f0 = jnp.where(ok0, e0v * CAP + s0, 0.0)
    f1 = jnp.where(ok1, e1v * CAP + s1, 0.0)
    g0v = jnp.where(ok0, g0, 0.0)
    g1v = jnp.where(ok1, g1, 0.0)
    pertok_ref[...] = (jnp.where(lane == 0, f0, 0.0)
                       + jnp.where(lane == 1, f1, 0.0)
                       + jnp.where(lane == 2, g0v, 0.0)
                       + jnp.where(lane == 3, g1v, 0.0))

    @pl.when(pid == 0)
    def _():
        idxf_ref[...] = jnp.zeros((E, 1, CAP), jnp.float32)

    rank_sel = jnp.where((is0 | is1), rank, -1.0)
    r_iota = jax.lax.broadcasted_iota(jnp.int32, (BR, CAP), 1)
    tok_row = (rstart + jax.lax.broadcasted_iota(
        jnp.int32, (1, BR), 1)).astype(jnp.float32)
    for e in range(E):
        col = rank_sel[:, e:e + 1]
        a = jnp.where(col.astype(jnp.int32) == r_iota, 1.0, 0.0)
        idxf_ref[e, 0] += jnp.dot(tok_row, a,
                                  preferred_element_type=jnp.float32,
                                  precision=jax.lax.Precision.HIGHEST)[0]


def _k5_body(xin_ref, w1_ref, b1_ref, w2_ref, b2_ref, oexp_ref, acc_scr):
    ffb = pl.program_id(1)

    @pl.when(ffb == 0)
    def _():
        acc_scr[...] = jnp.zeros((CAP, D), jnp.float32)

    mid = jnp.maximum(
        jnp.dot(xin_ref[...], w1_ref[0], preferred_element_type=jnp.float32)
        + b1_ref[0], 0.0)
    acc_scr[...] += jnp.dot(mid, w2_ref[0], preferred_element_type=jnp.float32)

    @pl.when(ffb == FF // BFF - 1)
    def _():
        oexp_ref[0] = acc_scr[...] + b2_ref[0]


def _make_sc_gather(rows_total, table_rows):
    info = plsc.get_sparse_core_info()
    nw = info.num_cores * info.num_subcores
    per_w = rows_total // nw
    chunk = min(64, per_w)
    n_iter = per_w // chunk
    mesh = plsc.VectorSubcoreMesh(core_axis_name="c", subcore_axis_name="s")

    @functools.partial(
        pl.kernel, mesh=mesh,
        out_type=jax.ShapeDtypeStruct((rows_total, D), jnp.float32),
        scratch_types=[pltpu.VMEM((chunk,), jnp.int32),
                       pltpu.VMEM((chunk, D), jnp.float32),
                       pltpu.SemaphoreType.DMA],
    )
    def g(table_hbm, idx_hbm, out_hbm, idx_v, rows_v, sem):
        wid = lax.axis_index("s") * info.num_cores + lax.axis_index("c")
        for c in range(n_iter):
            base = wid * per_w + c * chunk
            pltpu.sync_copy(idx_hbm.at[pl.ds(base, chunk)], idx_v)
            pltpu.async_copy(table_hbm.at[idx_v], rows_v, sem).wait()
            pltpu.sync_copy(rows_v, out_hbm.at[pl.ds(base, chunk)])

    return g


def _k6_body(pertok_ref, u0_ref, u1_ref, x1_ref, out_ref):
    g0 = pertok_ref[:, 2:3]
    g1 = pertok_ref[:, 3:4]
    out_ref[...] = x1_ref[...] + g0 * u0_ref[...] + g1 * u1_ref[...]


def kernel(x, Wqkv, Wproj, ln1_g, ln1_b, ln2_g, ln2_b, Wr, br, Wn, bn,
           We1, be1, We2, be2):
    f32 = jnp.float32
    x2 = x.reshape(N, D)

    half = DH // 2
    pos = jnp.arange(T, dtype=f32)[:, None]
    inv = jnp.exp(jnp.arange(0, DH, 2, dtype=f32) * (-math.log(10000.0) / DH))
    ang = pos * inv
    cos1 = jnp.cos(ang)
    sin1 = jnp.sin(ang)
    cos_full = jnp.tile(jnp.concatenate([cos1, cos1], axis=1), (1, H))
    sin_full = jnp.tile(jnp.concatenate([-sin1, sin1], axis=1), (1, H))

    eps = jax.random.normal(jax.random.key(42), (B, T, E), dtype=f32)
    eps_p = jnp.zeros((N, EP), f32).at[:, :E].set(eps.reshape(N, E))
    Wr_p = jnp.zeros((D, EP), f32).at[:, :E].set(Wr)
    Wn_p = jnp.zeros((D, EP), f32).at[:, :E].set(Wn)
    br_p = jnp.full((1, EP), NEG, f32).at[0, :E].set(br)
    bn_p = jnp.zeros((1, EP), f32).at[0, :E].set(bn)
    ln1g = ln1_g.reshape(1, D)
    ln1b = ln1_b.reshape(1, D)
    ln2g = ln2_g.reshape(1, D)
    ln2b = ln2_b.reshape(1, D)
    be1_3 = be1.reshape(E, 1, FF)
    be2_3 = be2.reshape(E, 1, D)

    row_spec = pl.BlockSpec((BT, D), lambda i: (i, 0))
    vec_spec = pl.BlockSpec((1, D), lambda i: (0, 0))
    q, k, v = pl.pallas_call(
        _k1_body,
        grid=(N // BT,),
        in_specs=[row_spec, vec_spec, vec_spec,
                  pl.BlockSpec((D, 3 * D), lambda i: (0, 0)),
                  row_spec, row_spec],
        out_specs=[row_spec, row_spec, row_spec],
        out_shape=[jax.ShapeDtypeStruct((N, D), f32)] * 3,
    )(x2, ln1g, ln1b, Wqkv, cos_full, sin_full)

    ctx = pl.pallas_call(
        _k2_body,
        grid=(H // 2, N // BQ),
        in_specs=[pl.BlockSpec((BQ, 2 * DH), lambda hp, qb: (qb, hp)),
                  pl.BlockSpec((N, 2 * DH), lambda hp, qb: (0, hp)),
                  pl.BlockSpec((N, 2 * DH), lambda hp, qb: (0, hp))],
        out_specs=pl.BlockSpec((2, BQ, DH), lambda hp, qb: (hp, qb, 0)),
        out_shape=jax.ShapeDtypeStruct((H, T, DH), f32),
    )(q, k, v)
    ctx = ctx.reshape(N, D)

    ep_spec = pl.BlockSpec((BT, EP), lambda i: (i, 0))
    ep_vec = pl.BlockSpec((1, EP), lambda i: (0, 0))
    x1, h2, noisy = pl.pallas_call(
        _k3_body,
        grid=(N // BT,),
        in_specs=[row_spec, row_spec,
                  pl.BlockSpec((D, D), lambda i: (0, 0)),
                  vec_spec, vec_spec,
                  pl.BlockSpec((D, EP), lambda i: (0, 0)), ep_vec,
                  pl.BlockSpec((D, EP), lambda i: (0, 0)), ep_vec,
                  ep_spec],
        out_specs=[row_spec, row_spec, ep_spec],
        out_shape=[jax.ShapeDtypeStruct((N, D), f32),
                   jax.ShapeDtypeStruct((N, D), f32),
                   jax.ShapeDtypeStruct((N, EP), f32)],
    )(x2, ctx, Wproj, ln2g, ln2b, Wr_p, br_p, Wn_p, bn_p, eps_p)

    BR = N // 16
    pertok, idxf = pl.pallas_call(
        _k4_body,
        grid=(16,),
        in_specs=[pl.BlockSpec((N, EP), lambda i: (0, 0))],
        out_specs=[pl.BlockSpec((BR, EP), lambda i: (i, 0)),
                   pl.BlockSpec((E, 1, CAP), lambda i: (0, 0, 0))],
        out_shape=[jax.ShapeDtypeStruct((N, EP), f32),
                   jax.ShapeDtypeStruct((E, 1, CAP), f32)],
    )(noisy)

    disp_idx = idxf.reshape(E * CAP).astype(jnp.int32)
    flat01 = jnp.concatenate(
        [pertok[:, 0], pertok[:, 1]]).astype(jnp.int32)

    xin = _make_sc_gather(E * CAP, N)(h2, disp_idx)

    oexp = pl.pallas_call(
        _k5_body,
        grid=(E, FF // BFF),
        in_specs=[pl.BlockSpec((CAP, D), lambda e, f: (e, 0)),
                  pl.BlockSpec((1, D, BFF), lambda e, f: (e, 0, f)),
                  pl.BlockSpec((1, 1, BFF), lambda e, f: (e, 0, f)),
                  pl.BlockSpec((1, BFF, D), lambda e, f: (e, f, 0)),
                  pl.BlockSpec((1, 1, D), lambda e, f: (e, 0, 0))],
        out_specs=pl.BlockSpec((1, CAP, D), lambda e, f: (e, 0, 0)),
        out_shape=jax.ShapeDtypeStruct((E, CAP, D), f32),
        scratch_shapes=[pltpu.VMEM((CAP, D), f32)],
    )(xin, We1, be1_3, We2, be2_3)

    u01 = _make_sc_gather(2 * N, E * CAP)(oexp.reshape(E * CAP, D), flat01)

    out = pl.pallas_call(
        _k6_body,
        grid=(N // BT,),
        in_specs=[pl.BlockSpec((BT, EP), lambda i: (i, 0)),
                  pl.BlockSpec((BT, D), lambda i: (i, 0)),
                  pl.BlockSpec((BT, D), lambda i: (i + N // BT, 0)),
                  pl.BlockSpec((BT, D), lambda i: (i, 0))],
        out_specs=pl.BlockSpec((BT, D), lambda i: (i, 0)),
        out_shape=jax.ShapeDtypeStruct((N, D), f32),
    )(pertok, u01, u01, x1)

    return out.reshape(B, T, D)


# speedup vs baseline: 1.4372x; 1.0034x over previous
import functools
import math

import jax
import jax.numpy as jnp
from jax import lax
from jax.experimental import pallas as pl
from jax.experimental.pallas import tpu as pltpu
from jax.experimental.pallas import tpu_sc as plsc

B, T, D, H, E, K = 1, 2048, 1024, 16, 8, 2
DH = D // H
FF = 4 * D
N = B * T
CAP = N * K // E
EP = 128
NEG = -1e30

BT = 512
BQ = 1024
BK = 1024
BFF = 2048


def _k1_body(x_ref, g_ref, b_ref, wqkv_ref, cos_ref, sin_ref,
             q_ref, k_ref, v_ref):
    x = x_ref[...]
    mu = jnp.mean(x, axis=1, keepdims=True)
    var = jnp.mean((x - mu) ** 2, axis=1, keepdims=True)
    h = (x - mu) / jnp.sqrt(var + 1e-5) * g_ref[...] + b_ref[...]
    qkv = jnp.dot(h, wqkv_ref[...], preferred_element_type=jnp.float32)
    q = qkv[:, :D]
    k = qkv[:, D:2 * D]
    v = qkv[:, 2 * D:]
    cos = cos_ref[...]
    sin = sin_ref[...]
    lane = jax.lax.broadcasted_iota(jnp.int32, (BT, D), 1)
    first_half = (lane % DH) < (DH // 2)

    def rot(a):
        a_sw = jnp.where(first_half,
                         jnp.roll(a, -DH // 2, axis=1),
                         jnp.roll(a, DH // 2, axis=1))
        return a * cos + a_sw * sin

    q_ref[...] = rot(q)
    k_ref[...] = rot(k)
    v_ref[...] = v


def _k2_body(q_ref, k_ref, v_ref, o_ref):
    qb = pl.program_id(1)
    scale = 1.0 / math.sqrt(DH)
    rows = qb * BQ + jax.lax.broadcasted_iota(jnp.int32, (BQ, BK), 0)

    for sub in range(2):
        q = q_ref[:, sub * DH:(sub + 1) * DH] * scale

        def body(kb, carry):
            m, l, acc = carry
            kk = k_ref[pl.ds(kb * BK, BK), sub * DH:(sub + 1) * DH]
            vv = v_ref[pl.ds(kb * BK, BK), sub * DH:(sub + 1) * DH]
            s = jax.lax.dot_general(q, kk, (((1,), (1,)), ((), ())),
                                    preferred_element_type=jnp.float32)
            cols = kb * BK + jax.lax.broadcasted_iota(jnp.int32, (BQ, BK), 1)
            s = jnp.where(rows >= cols, s, NEG)
            m_new = jnp.maximum(m, jnp.max(s, axis=1, keepdims=True))
            p = jnp.exp(s - m_new)
            corr = jnp.exp(m - m_new)
            l_new = l * corr + jnp.sum(p, axis=1, keepdims=True)
            acc_new = acc * corr + jnp.dot(p, vv,
                                           preferred_element_type=jnp.float32)
            return m_new, l_new, acc_new

        m0 = jnp.full((BQ, 1), NEG, jnp.float32)
        l0 = jnp.zeros((BQ, 1), jnp.float32)
        a0 = jnp.zeros((BQ, DH), jnp.float32)
        m, l, acc = jax.lax.fori_loop(0, qb + 1, body, (m0, l0, a0))
        o_ref[sub] = acc / l


def _k3_body(x_ref, ctx_ref, wproj_ref, g_ref, b_ref,
             wr_ref, br_ref, wn_ref, bn_ref, eps_ref,
             x1_ref, h2_ref, noisy_ref):
    x1 = x_ref[...] + jnp.dot(ctx_ref[...], wproj_ref[...],
                              preferred_element_type=jnp.float32)
    x1_ref[...] = x1
    mu = jnp.mean(x1, axis=1, keepdims=True)
    var = jnp.mean((x1 - mu) ** 2, axis=1, keepdims=True)
    h2 = (x1 - mu) / jnp.sqrt(var + 1e-5) * g_ref[...] + b_ref[...]
    h2_ref[...] = h2
    logits = jnp.dot(h2, wr_ref[...], preferred_element_type=jnp.float32) + br_ref[...]
    pre = jnp.dot(h2, wn_ref[...], preferred_element_type=jnp.float32) + bn_ref[...]
    noise = jnp.maximum(pre, 0.0) + jnp.log1p(jnp.exp(-jnp.abs(pre)))
    noisy_ref[...] = logits + eps_ref[...] * noise


def _top2(nz, rows_n):
    lane = jax.lax.broadcasted_iota(jnp.int32, (rows_n, EP), 1)
    v0 = jnp.max(nz, axis=1, keepdims=True)
    e0 = jnp.min(jnp.where(nz == v0, lane, EP), axis=1, keepdims=True)
    nz1 = jnp.where(lane == e0, NEG, nz)
    v1 = jnp.max(nz1, axis=1, keepdims=True)
    e1 = jnp.min(jnp.where(nz1 == v1, lane, EP), axis=1, keepdims=True)
    is0 = (lane == e0)
    is1 = (lane == e1)
    mask = jnp.where(is0 | is1, 1.0, 0.0)
    ev = jnp.exp(v1 - v0)
    g0 = 1.0 / (1.0 + ev)
    g1 = ev / (1.0 + ev)
    return mask, is0, is1, g0, g1, lane


def _k4_body(noisy_ref, pertok_ref, idxf_ref):
    pid = pl.program_id(0)
    BR = N // 16
    rstart = pid * BR

    mask, _, _, _, _, _ = _top2(noisy_ref[...], N)
    _, is0, is1, g0, g1, lane = _top2(noisy_ref[pl.ds(rstart, BR), :], BR)

    rows = rstart + jax.lax.broadcasted_iota(jnp.int32, (BR, N), 0)
    tcols = jax.lax.broadcasted_iota(jnp.int32, (BR, N), 1)
    lt = jnp.where(tcols < rows, 1.0, 0.0)
    rank = jnp.dot(lt, mask, preferred_element_type=jnp.float32)
    lane_f = lane.astype(jnp.float32)

    e0v = jnp.sum(jnp.where(is0, lane_f, 0.0), axis=1, keepdims=True)
    e1v = jnp.sum(jnp.where(is1, lane_f, 0.0), axis=1, keepdims=True)
    s0 = jnp.sum(jnp.where(is0, rank, 0.0), axis=1, keepdims=True)
    s1 = jnp.sum(jnp.where(is1, rank, 0.0), axis=1, keepdims=True)
    ok0 = s0 < CAP
    ok1 = s1 < CAP
    f0 = jnp.where(ok0, e0v * CAP + s0, 0.0)
    f1 = jnp.where(ok1, e1v * CAP + s1, 0.0)
    g0v = jnp.where(ok0, g0, 0.0)
    g1v = jnp.where(ok1, g1, 0.0)
    pertok_ref[...] = (jnp.where(lane == 0, f0, 0.0)
                       + jnp.where(lane == 1, f1, 0.0)
                       + jnp.where(lane == 2, g0v, 0.0)
                       + jnp.where(lane == 3, g1v, 0.0))

    @pl.when(pid == 0)
    def _():
        idxf_ref[...] = jnp.zeros((E, 1, CAP), jnp.float32)

    rank_sel = jnp.where((is0 | is1), rank, -1.0)
    r_iota = jax.lax.broadcasted_iota(jnp.int32, (BR, CAP), 1)
    tok_row = (rstart + jax.lax.broadcasted_iota(
        jnp.int32, (1, BR), 1)).astype(jnp.float32)
    for e in range(E):
        col = rank_sel[:, e:e + 1]
        a = jnp.where(col.astype(jnp.int32) == r_iota, 1.0, 0.0)
        idxf_ref[e, 0] += jnp.dot(tok_row, a,
                                  preferred_element_type=jnp.float32,
                                  precision=jax.lax.Precision.HIGHEST)[0]


def _k5_body(xin_ref, w1_ref, b1_ref, w2_ref, b2_ref, oexp_ref, acc_scr):
    ffb = pl.program_id(1)

    @pl.when(ffb == 0)
    def _():
        acc_scr[...] = jnp.zeros((CAP, D), jnp.float32)

    mid = jnp.maximum(
        jnp.dot(xin_ref[...], w1_ref[0], preferred_element_type=jnp.float32)
        + b1_ref[0], 0.0)
    acc_scr[...] += jnp.dot(mid, w2_ref[0], preferred_element_type=jnp.float32)

    @pl.when(ffb == FF // BFF - 1)
    def _():
        oexp_ref[0] = acc_scr[...] + b2_ref[0]


def _make_sc_gather(rows_total, table_rows):
    info = plsc.get_sparse_core_info()
    nw = info.num_cores * info.num_subcores
    per_w = rows_total // nw
    chunk = min(64, per_w)
    n_iter = per_w // chunk
    mesh = plsc.VectorSubcoreMesh(core_axis_name="c", subcore_axis_name="s")

    @functools.partial(
        pl.kernel, mesh=mesh,
        out_type=jax.ShapeDtypeStruct((rows_total, D), jnp.float32),
        scratch_types=[pltpu.VMEM((chunk,), jnp.int32),
                       pltpu.VMEM((chunk, D), jnp.float32),
                       pltpu.SemaphoreType.DMA],
    )
    def g(table_hbm, idx_hbm, out_hbm, idx_v, rows_v, sem):
        wid = lax.axis_index("s") * info.num_cores + lax.axis_index("c")
        for c in range(n_iter):
            base = wid * per_w + c * chunk
            pltpu.sync_copy(idx_hbm.at[pl.ds(base, chunk)], idx_v)
            pltpu.async_copy(table_hbm.at[idx_v], rows_v, sem).wait()
            pltpu.sync_copy(rows_v, out_hbm.at[pl.ds(base, chunk)])

    return g


def _k6_body(pertok_ref, u0_ref, u1_ref, x1_ref, out_ref):
    g0 = pertok_ref[:, 2:3]
    g1 = pertok_ref[:, 3:4]
    out_ref[...] = x1_ref[...] + g0 * u0_ref[...] + g1 * u1_ref[...]


def kernel(x, Wqkv, Wproj, ln1_g, ln1_b, ln2_g, ln2_b, Wr, br, Wn, bn,
           We1, be1, We2, be2):
    f32 = jnp.float32
    x2 = x.reshape(N, D)

    half = DH // 2
    pos = jnp.arange(T, dtype=f32)[:, None]
    inv = jnp.exp(jnp.arange(0, DH, 2, dtype=f32) * (-math.log(10000.0) / DH))
    ang = pos * inv
    cos1 = jnp.cos(ang)
    sin1 = jnp.sin(ang)
    cos_full = jnp.tile(jnp.concatenate([cos1, cos1], axis=1), (1, H))
    sin_full = jnp.tile(jnp.concatenate([-sin1, sin1], axis=1), (1, H))

    eps = jax.random.normal(jax.random.key(42), (B, T, E), dtype=f32)
    eps_p = jnp.zeros((N, EP), f32).at[:, :E].set(eps.reshape(N, E))
    Wr_p = jnp.zeros((D, EP), f32).at[:, :E].set(Wr)
    Wn_p = jnp.zeros((D, EP), f32).at[:, :E].set(Wn)
    br_p = jnp.full((1, EP), NEG, f32).at[0, :E].set(br)
    bn_p = jnp.zeros((1, EP), f32).at[0, :E].set(bn)
    ln1g = ln1_g.reshape(1, D)
    ln1b = ln1_b.reshape(1, D)
    ln2g = ln2_g.reshape(1, D)
    ln2b = ln2_b.reshape(1, D)
    be1_3 = be1.reshape(E, 1, FF)
    be2_3 = be2.reshape(E, 1, D)

    row_spec = pl.BlockSpec((BT, D), lambda i: (i, 0))
    vec_spec = pl.BlockSpec((1, D), lambda i: (0, 0))
    q, k, v = pl.pallas_call(
        _k1_body,
        grid=(N // BT,),
        in_specs=[row_spec, vec_spec, vec_spec,
                  pl.BlockSpec((D, 3 * D), lambda i: (0, 0)),
                  row_spec, row_spec],
        out_specs=[row_spec, row_spec, row_spec],
        out_shape=[jax.ShapeDtypeStruct((N, D), f32)] * 3,
    )(x2, ln1g, ln1b, Wqkv, cos_full, sin_full)

    ctx = pl.pallas_call(
        _k2_body,
        grid=(H // 2, N // BQ),
        in_specs=[pl.BlockSpec((BQ, 2 * DH), lambda hp, qb: (qb, hp)),
                  pl.BlockSpec((N, 2 * DH), lambda hp, qb: (0, hp)),
                  pl.BlockSpec((N, 2 * DH), lambda hp, qb: (0, hp))],
        out_specs=pl.BlockSpec((2, BQ, DH), lambda hp, qb: (hp, qb, 0)),
        out_shape=jax.ShapeDtypeStruct((H, T, DH), f32),
    )(q, k, v)
    ctx = ctx.reshape(N, D)

    ep_spec = pl.BlockSpec((BT, EP), lambda i: (i, 0))
    ep_vec = pl.BlockSpec((1, EP), lambda i: (0, 0))
    x1, h2, noisy = pl.pallas_call(
        _k3_body,
        grid=(N // BT,),
        in_specs=[row_spec, row_spec,
                  pl.BlockSpec((D, D), lambda i: (0, 0)),
                  vec_spec, vec_spec,
                  pl.BlockSpec((D, EP), lambda i: (0, 0)), ep_vec,
                  pl.BlockSpec((D, EP), lambda i: (0, 0)), ep_vec,
                  ep_spec],
        out_specs=[row_spec, row_spec, ep_spec],
        out_shape=[jax.ShapeDtypeStruct((N, D), f32),
                   jax.ShapeDtypeStruct((N, D), f32),
                   jax.ShapeDtypeStruct((N, EP), f32)],
    )(x2, ctx, Wproj, ln2g, ln2b, Wr_p, br_p, Wn_p, bn_p, eps_p)

    BR = N // 16
    pertok, idxf = pl.pallas_call(
        _k4_body,
        grid=(16,),
        in_specs=[pl.BlockSpec((N, EP), lambda i: (0, 0))],
        out_specs=[pl.BlockSpec((BR, EP), lambda i: (i, 0)),
                   pl.BlockSpec((E, 1, CAP), lambda i: (0, 0, 0))],
        out_shape=[jax.ShapeDtypeStruct((N, EP), f32),
                   jax.ShapeDtypeStruct((E, 1, CAP), f32)],
    )(noisy)

    disp_idx = idxf.reshape(E * CAP).astype(jnp.int32)
    flat01 = jnp.concatenate(
        [pertok[:, 0], pertok[:, 1]]).astype(jnp.int32)

    xin = _make_sc_gather(E * CAP, N)(h2, disp_idx)

    oexp = pl.pallas_call(
        _k5_body,
        grid=(E, FF // BFF),
        in_specs=[pl.BlockSpec((CAP, D), lambda e, f: (e, 0)),
                  pl.BlockSpec((1, D, BFF), lambda e, f: (e, 0, f)),
                  pl.BlockSpec((1, 1, BFF), lambda e, f: (e, 0, f)),
                  pl.BlockSpec((1, BFF, D), lambda e, f: (e, f, 0)),
                  pl.BlockSpec((1, 1, D), lambda e, f: (e, 0, 0))],
        out_specs=pl.BlockSpec((1, CAP, D), lambda e, f: (e, 0, 0)),
        out_shape=jax.ShapeDtypeStruct((E, CAP, D), f32),
        scratch_shapes=[pltpu.VMEM((CAP, D), f32)],
    )(xin, We1, be1_3, We2, be2_3)

    u01 = _make_sc_gather(2 * N, E * CAP)(oexp.reshape(E * CAP, D), flat01)

    out = pl.pallas_call(
        _k6_body,
        grid=(N // BT,),
        in_specs=[pl.BlockSpec((BT, EP), lambda i: (i, 0)),
                  pl.BlockSpec((BT, D), lambda i: (i, 0)),
                  pl.BlockSpec((BT, D), lambda i: (i + N // BT, 0)),
                  pl.BlockSpec((BT, D), lambda i: (i, 0))],
        out_specs=pl.BlockSpec((BT, D), lambda i: (i, 0)),
        out_shape=jax.ShapeDtypeStruct((N, D), f32),
    )(pertok, u01, u01, x1)

    return out.reshape(B, T, D)
